# baseline (speedup 1.0000x reference)
import functools

import jax
import jax.numpy as jnp
from jax import lax
from jax.experimental import pallas as pl
from jax.experimental.pallas import tpu as pltpu
from jax.experimental.pallas import tpu_sc as plsc

N = 10000
E = 320000
D = 128
NGRAPH = 128
BN_EPS = 1e-5

NC = 2
NS = 16
NW = NC * NS
EPW = E // NW
CHUNK = 128
NCHUNKP = 79
EPWP = NCHUNKP * CHUNK
ZROWS = 80
NBLK = N // ZROWS
ACCR = N + ZROWS
NBLKZ = ACCR // ZROWS


def _sc_aggregate(srcp, dstp, h):
    mesh = plsc.VectorSubcoreMesh(
        core_axis_name="c", subcore_axis_name="s", num_cores=NC, num_subcores=NS
    )

    @functools.partial(
        pl.kernel,
        out_type=jax.ShapeDtypeStruct((NC * N, D), jnp.float32),
        mesh=mesh,
        scratch_types=[
            pltpu.VMEM((EPWP,), jnp.int32),
            [pltpu.VMEM((CHUNK,), jnp.int32) for _ in range(2)],
            [pltpu.VMEM((CHUNK, D), jnp.float32) for _ in range(2)],
            pltpu.VMEM_SHARED((ACCR, D), jnp.float32),
            [pltpu.SemaphoreType.DMA for _ in range(2)],
            [pltpu.SemaphoreType.DMA for _ in range(2)],
            [pltpu.SemaphoreType.DMA for _ in range(2)],
        ],
    )
    def agg(src_hbm, dst_hbm, h_hbm, out_hbm, idx_s, dbuf, rows,
            acc, semg, semd, sems):
        cid = lax.axis_index("c")
        sid = lax.axis_index("s")
        wid = sid * NC + cid
        nblkz = (NBLKZ - 1 - sid) // NS + 1
        nblk = (NBLK - 1 - sid) // NS + 1

        def zrow(i, _):
            for j in range(D // 16):
                rows[0][i, pl.ds(j * 16, 16)] = jnp.zeros((16,), jnp.float32)
            return 0

        lax.fori_loop(0, ZROWS, zrow, 0)

        def zacc(i, _):
            pltpu.sync_copy(rows[0].at[pl.ds(0, ZROWS)],
                            acc.at[pl.ds((sid + i * NS) * ZROWS, ZROWS)])
            return 0

        lax.fori_loop(0, nblkz, zacc, 0)

        pltpu.sync_copy(src_hbm.at[pl.ds(wid * EPWP, EPWP)], idx_s)
        plsc.subcore_barrier()

        def _g_start(i, b):
            pltpu.async_copy(h_hbm.at[idx_s.at[pl.ds(i * CHUNK, CHUNK)]],
                             rows[b], semg[b])

        def _g_wait(i, b):
            pltpu.make_async_copy(h_hbm.at[idx_s.at[pl.ds(i * CHUNK, CHUNK)]],
                                  rows[b], semg[b]).wait()

        def _d_start(i, b):
            pltpu.async_copy(dst_hbm.at[pl.ds(wid * EPWP + i * CHUNK, CHUNK)],
                             dbuf[b], semd[b])

        def _d_wait(i, b):
            pltpu.make_async_copy(dst_hbm.at[pl.ds(wid * EPWP + i * CHUNK, CHUNK)],
                                  dbuf[b], semd[b]).wait()

        def _s_start(b):
            pltpu.async_copy(rows[b], acc.at[dbuf[b]], sems[b], add=True)

        def _s_wait(b):
            pltpu.make_async_copy(rows[b], acc.at[dbuf[b]], sems[b]).wait()

        _g_start(0, 0)
        _d_start(0, 0)

        def pair(g, _):
            for k in range(2):
                i = 2 * g + k
                _g_wait(i, k)
                _d_wait(i, k)
                _s_start(k)
                if k == 0:
                    @pl.when(g > 0)
                    def _():
                        _s_wait(1)
                else:
                    _s_wait(0)
                _g_start(i + 1, 1 - k)
                _d_start(i + 1, 1 - k)
            return 0

        lax.fori_loop(0, (NCHUNKP - 1) // 2, pair, 0)
        last = NCHUNKP - 1
        _g_wait(last, 0)
        _d_wait(last, 0)
        _s_start(0)
        _s_wait(1)
        _s_wait(0)
        plsc.subcore_barrier()

        def wb(i, _):
            r0 = (sid + i * NS) * ZROWS
            pltpu.sync_copy(
                acc.at[pl.ds(r0, ZROWS)], out_hbm.at[pl.ds(cid * N + r0, ZROWS)]
            )
            return 0

        lax.fori_loop(0, nblk, wb, 0)

    out = agg(srcp, dstp, h)
    return out[:N], out[N:]


_RSQ = 1.0 / (1.0 + BN_EPS) ** 0.5


def _dot(a, b):
    return jnp.dot(a, b, preferred_element_type=jnp.float32)


def _dot_hi(a, b):
    return jnp.dot(a, b, preferred_element_type=jnp.float32,
                   precision=jax.lax.Precision.HIGHEST)


def _tc_gin(h, agg_a, agg_b, w1, b1, g1, be1, w2, b2):
    def body(h_ref, a_ref, c_ref, w1_ref, b1_ref, g1_ref, be1_ref,
             w2_ref, b2_ref, o_ref):
        u = h_ref[...] + a_ref[...] + c_ref[...]
        t = _dot(u, w1_ref[...]) + b1_ref[...]
        t = jax.nn.relu(t * _RSQ * g1_ref[...] + be1_ref[...])
        o_ref[...] = jax.nn.relu(_dot(t, w2_ref[...]) + b2_ref[...])

    r = lambda v: v.reshape(1, -1)
    return pl.pallas_call(
        body,
        out_shape=jax.ShapeDtypeStruct((h.shape[0], w2.shape[1]), jnp.float32),
    )(h, agg_a, agg_b, w1, r(b1), r(g1), r(be1), w2, r(b2))


def _tc_pool_head(h1, h2, h3, batch2d, w0s, head):
    wa, wb, wc = w0s

    def body(h1_ref, h2_ref, h3_ref, bt_ref, wa_ref, wb_ref, wc_ref,
             *rest_refs):
        o_ref = rest_refs[-1]
        rest = rest_refs[:-1]
        gids = lax.broadcasted_iota(jnp.int32, (NGRAPH, N), 0)
        oh = (bt_ref[...] == gids).astype(jnp.float32)
        p1 = _dot_hi(oh, h1_ref[...])
        p2 = _dot_hi(oh, h2_ref[...])
        p3 = _dot_hi(oh, h3_ref[...])
        g = _dot(p1, wa_ref[...]) + _dot(p2, wb_ref[...]) + _dot(p3, wc_ref[...])
        k = 0
        for li, (_, _, gg, _) in enumerate(head):
            if li > 0:
                w = rest[k]; k += 1
                g = _dot(g, w[...])
            b = rest[k]; k += 1
            g = g + b[...]
            if gg is not None:
                gref = rest[k]; beref = rest[k + 1]; k += 2
                g = jax.nn.relu(g * _RSQ * gref[...] + beref[...])
        o_ref[...] = g

    r = lambda v: v.reshape(1, -1)
    flat = []
    for li, (w, b, g, be) in enumerate(head):
        if li > 0:
            flat.append(w)
        flat.append(r(b))
        if g is not None:
            flat += [r(g), r(be)]
    return pl.pallas_call(
        body,
        out_shape=jax.ShapeDtypeStruct((NGRAPH, head[-1][0].shape[1]), jnp.float32),
    )(h1, h2, h3, batch2d, wa, wb, wc, *flat)


def _pad_cols(m, width):
    return jnp.pad(m, ((0, 0), (0, width - m.shape[1])))


def _pad_rows(m, height):
    return jnp.pad(m, ((0, height - m.shape[0]), (0, 0)))


def kernel(x, edge_index, batch, params):
    pad = EPWP - EPW
    srcp = jnp.pad(edge_index[0].reshape(NW, EPW),
                   ((0, 0), (0, pad))).reshape(-1)
    dstp = jnp.pad(edge_index[1].reshape(NW, EPW), ((0, 0), (0, pad)),
                   constant_values=N).reshape(-1)
    gin = params["gin"]

    w2 = _pad_cols(gin[0]["W2"], D)
    b2 = jnp.pad(gin[0]["b2"], (0, D - 64))
    agg_a, agg_b = _sc_aggregate(srcp, dstp, x)
    h1 = _tc_gin(x, agg_a, agg_b, gin[0]["W1"], gin[0]["b1"],
                 gin[0]["g"], gin[0]["be"], w2, b2)

    agg_a, agg_b = _sc_aggregate(srcp, dstp, h1)
    h2 = _tc_gin(h1, agg_a, agg_b, _pad_rows(gin[1]["W1"], D), gin[1]["b1"],
                 gin[1]["g"], gin[1]["be"], gin[1]["W2"], gin[1]["b2"])

    agg_a, agg_b = _sc_aggregate(srcp, dstp, h2)
    h3 = _tc_gin(h2, agg_a, agg_b, gin[2]["W1"], gin[2]["b1"],
                 gin[2]["g"], gin[2]["be"], gin[2]["W2"], gin[2]["b2"])

    mlp = params["mlp"]
    w0 = mlp[0]["W"]
    w0s = (_pad_rows(w0[:64], D), w0[64:192], w0[192:])
    head = [(w0, mlp[0]["b"], mlp[0]["g"], mlp[0]["be"])]
    for li in range(1, len(mlp)):
        head.append((mlp[li]["W"], mlp[li]["b"],
                     mlp[li].get("g"), mlp[li].get("be")))
    batch2d = batch.reshape(1, N)
    return _tc_pool_head(h1, h2, h3, batch2d, w0s, head)


# speedup vs baseline: 1.0779x; 1.0779x over previous
import functools

import jax
import jax.numpy as jnp
from jax import lax
from jax.experimental import pallas as pl
from jax.experimental.pallas import tpu as pltpu
from jax.experimental.pallas import tpu_sc as plsc

N = 10000
E = 320000
D = 128
NGRAPH = 128
BN_EPS = 1e-5

NC = 2
NS = 16
NW = NC * NS
EPW = E // NW
CHUNK = 128
NCHUNKP = 79
EPWP = NCHUNKP * CHUNK
ZROWS = 80
NBLK = N // ZROWS
JUNK = 16
ACCR = N + JUNK


def _sc_aggregate(srcp, dstp, h):
    mesh = plsc.VectorSubcoreMesh(
        core_axis_name="c", subcore_axis_name="s", num_cores=NC, num_subcores=NS
    )

    @functools.partial(
        pl.kernel,
        out_type=jax.ShapeDtypeStruct((NC * N, D), jnp.float32),
        mesh=mesh,
        scratch_types=[
            [pltpu.VMEM((CHUNK,), jnp.int32) for _ in range(3)],
            [pltpu.VMEM((CHUNK,), jnp.int32) for _ in range(3)],
            [pltpu.VMEM((CHUNK, D), jnp.float32) for _ in range(3)],
            pltpu.VMEM_SHARED((ACCR, D), jnp.float32),
            [pltpu.SemaphoreType.DMA for _ in range(3)],
            [pltpu.SemaphoreType.DMA for _ in range(3)],
            [pltpu.SemaphoreType.DMA for _ in range(3)],
            [pltpu.SemaphoreType.DMA for _ in range(3)],
        ],
    )
    def agg(src_hbm, dst_hbm, h_hbm, out_hbm, sbuf, dbuf, rows,
            acc, semg, semd, sems, semsrc):
        cid = lax.axis_index("c")
        sid = lax.axis_index("s")
        wid = sid * NC + cid
        nblk = (NBLK - 1 - sid) // NS + 1

        def zrow(i, _):
            for j in range(D // 16):
                rows[0][i, pl.ds(j * 16, 16)] = jnp.zeros((16,), jnp.float32)
            return 0

        lax.fori_loop(0, ZROWS, zrow, 0)

        def zacc(i, _):
            pltpu.sync_copy(rows[0].at[pl.ds(0, ZROWS)],
                            acc.at[pl.ds((sid + i * NS) * ZROWS, ZROWS)])
            return 0

        lax.fori_loop(0, nblk, zacc, 0)

        @pl.when(sid == 0)
        def _():
            pltpu.sync_copy(rows[0].at[pl.ds(0, JUNK)],
                            acc.at[pl.ds(N, JUNK)])

        plsc.subcore_barrier()

        def _g_start(b):
            pltpu.async_copy(h_hbm.at[sbuf[b]], rows[b], semg[b])

        def _g_wait(b):
            pltpu.make_async_copy(h_hbm.at[sbuf[b]], rows[b], semg[b]).wait()

        def _i_start(i, b):
            pltpu.async_copy(src_hbm.at[pl.ds(wid * EPWP + i * CHUNK, CHUNK)],
                             sbuf[b], semsrc[b])
            pltpu.async_copy(dst_hbm.at[pl.ds(wid * EPWP + i * CHUNK, CHUNK)],
                             dbuf[b], semd[b])

        def _i_wait(i, b):
            pltpu.make_async_copy(src_hbm.at[pl.ds(wid * EPWP + i * CHUNK, CHUNK)],
                                  sbuf[b], semsrc[b]).wait()
            pltpu.make_async_copy(dst_hbm.at[pl.ds(wid * EPWP + i * CHUNK, CHUNK)],
                                  dbuf[b], semd[b]).wait()

        def _s_start(b):
            pltpu.async_copy(rows[b], acc.at[dbuf[b]], sems[b], add=True)

        def _s_wait(b):
            pltpu.make_async_copy(rows[b], acc.at[dbuf[b]], sems[b]).wait()

        _i_start(0, 0)
        _i_start(1, 1)
        _i_wait(0, 0)
        _g_start(0)

        def triple(g, _):
            i0 = 3 * g
            for k in range(3):
                i = i0 + k
                bp = (k + 2) % 3
                if k == 0:
                    @pl.when(g * 3 + 1 < NCHUNKP)
                    def _():
                        _i_wait(i + 1, (k + 1) % 3)
                        _g_start((k + 1) % 3)
                else:
                    _i_wait(i + 1, (k + 1) % 3)
                    _g_start((k + 1) % 3)
                _g_wait(k)
                _s_start(k)
                if k == 0:
                    @pl.when(g > 0)
                    def _():
                        _s_wait(bp)
                else:
                    _s_wait(bp)
                _i_start(i + 2, bp)
            return 0

        lax.fori_loop(0, 25, triple, 0)
        for i in range(75, NCHUNKP):
            b = i % 3
            bp = (b + 2) % 3
            if i + 1 < NCHUNKP:
                _i_wait(i + 1, (b + 1) % 3)
                _g_start((b + 1) % 3)
            _g_wait(b)
            _s_start(b)
            _s_wait(bp)
            if i + 2 < NCHUNKP:
                _i_start(i + 2, bp)
        _s_wait((NCHUNKP - 1) % 3)
        plsc.subcore_barrier()

        def wb(i, _):
            r0 = (sid + i * NS) * ZROWS
            pltpu.sync_copy(
                acc.at[pl.ds(r0, ZROWS)], out_hbm.at[pl.ds(cid * N + r0, ZROWS)]
            )
            return 0

        lax.fori_loop(0, nblk, wb, 0)

    out = agg(srcp, dstp, h)
    return out[:N], out[N:]


_RSQ = 1.0 / (1.0 + BN_EPS) ** 0.5


def _dot(a, b):
    return jnp.dot(a, b, preferred_element_type=jnp.float32)


def _dot_hi(a, b):
    return jnp.dot(a, b, preferred_element_type=jnp.float32,
                   precision=jax.lax.Precision.HIGHEST)


def _tc_gin(h, agg_a, agg_b, w1, b1, g1, be1, w2, b2):
    def body(h_ref, a_ref, c_ref, w1_ref, b1_ref, g1_ref, be1_ref,
             w2_ref, b2_ref, o_ref):
        u = h_ref[...] + a_ref[...] + c_ref[...]
        t = _dot(u, w1_ref[...]) + b1_ref[...]
        t = jax.nn.relu(t * _RSQ * g1_ref[...] + be1_ref[...])
        o_ref[...] = jax.nn.relu(_dot(t, w2_ref[...]) + b2_ref[...])

    r = lambda v: v.reshape(1, -1)
    return pl.pallas_call(
        body,
        out_shape=jax.ShapeDtypeStruct((h.shape[0], w2.shape[1]), jnp.float32),
    )(h, agg_a, agg_b, w1, r(b1), r(g1), r(be1), w2, r(b2))


def _tc_pool_head(h1, h2, h3, batch2d, w0s, head):
    wa, wb, wc = w0s

    def body(h1_ref, h2_ref, h3_ref, bt_ref, wa_ref, wb_ref, wc_ref,
             *rest_refs):
        o_ref = rest_refs[-1]
        rest = rest_refs[:-1]
        gids = lax.broadcasted_iota(jnp.int32, (NGRAPH, N), 0)
        oh = (bt_ref[...] == gids).astype(jnp.float32)
        p1 = _dot_hi(oh, h1_ref[...])
        p2 = _dot_hi(oh, h2_ref[...])
        p3 = _dot_hi(oh, h3_ref[...])
        g = _dot(p1, wa_ref[...]) + _dot(p2, wb_ref[...]) + _dot(p3, wc_ref[...])
        k = 0
        for li, (_, _, gg, _) in enumerate(head):
            if li > 0:
                w = rest[k]; k += 1
                g = _dot(g, w[...])
            b = rest[k]; k += 1
            g = g + b[...]
            if gg is not None:
                gref = rest[k]; beref = rest[k + 1]; k += 2
                g = jax.nn.relu(g * _RSQ * gref[...] + beref[...])
        o_ref[...] = g

    r = lambda v: v.reshape(1, -1)
    flat = []
    for li, (w, b, g, be) in enumerate(head):
        if li > 0:
            flat.append(w)
        flat.append(r(b))
        if g is not None:
            flat += [r(g), r(be)]
    return pl.pallas_call(
        body,
        out_shape=jax.ShapeDtypeStruct((NGRAPH, head[-1][0].shape[1]), jnp.float32),
    )(h1, h2, h3, batch2d, wa, wb, wc, *flat)


def _pad_cols(m, width):
    return jnp.pad(m, ((0, 0), (0, width - m.shape[1])))


def _pad_rows(m, height):
    return jnp.pad(m, ((0, height - m.shape[0]), (0, 0)))


def kernel(x, edge_index, batch, params):
    pad = EPWP - EPW
    srcp = jnp.pad(edge_index[0].reshape(NW, EPW),
                   ((0, 0), (0, pad))).reshape(-1)
    dstp = jnp.pad(edge_index[1].reshape(NW, EPW), ((0, 0), (0, pad)),
                   constant_values=N).reshape(-1)
    gin = params["gin"]

    w2 = _pad_cols(gin[0]["W2"], D)
    b2 = jnp.pad(gin[0]["b2"], (0, D - 64))
    agg_a, agg_b = _sc_aggregate(srcp, dstp, x)
    h1 = _tc_gin(x, agg_a, agg_b, gin[0]["W1"], gin[0]["b1"],
                 gin[0]["g"], gin[0]["be"], w2, b2)

    agg_a, agg_b = _sc_aggregate(srcp, dstp, h1)
    h2 = _tc_gin(h1, agg_a, agg_b, _pad_rows(gin[1]["W1"], D), gin[1]["b1"],
                 gin[1]["g"], gin[1]["be"], gin[1]["W2"], gin[1]["b2"])

    agg_a, agg_b = _sc_aggregate(srcp, dstp, h2)
    h3 = _tc_gin(h2, agg_a, agg_b, gin[2]["W1"], gin[2]["b1"],
                 gin[2]["g"], gin[2]["be"], gin[2]["W2"], gin[2]["b2"])

    mlp = params["mlp"]
    w0 = mlp[0]["W"]
    w0s = (_pad_rows(w0[:64], D), w0[64:192], w0[192:])
    head = [(w0, mlp[0]["b"], mlp[0]["g"], mlp[0]["be"])]
    for li in range(1, len(mlp)):
        head.append((mlp[li]["W"], mlp[li]["b"],
                     mlp[li].get("g"), mlp[li].get("be")))
    batch2d = batch.reshape(1, N)
    return _tc_pool_head(h1, h2, h3, batch2d, w0s, head)
